# baseline (speedup 1.0000x reference)
import functools

import numpy as np

import jax
import jax.numpy as jnp
from jax import lax
from jax.experimental import pallas as pl
from jax.experimental.pallas import tpu as pltpu
from jax.experimental.pallas import tpu_sc as plsc

B = 32
P = 16
SG = 1000
NSAMP = 150
GRID = 32
EPS = 1e-12
BIG = 1e4

L = 16
SG_PAD = 1008
NS_PAD = 160
U1 = 7
N_CH1 = SG_PAD // L
N_CH2 = NS_PAD // L
NCELL = GRID * GRID * GRID

OFF_SHAPE = 0
OFF_TRANS = 48
OFF_QUAT = 96
OFF_IU = 160
OFF_PTS = 176
ROW = 3200

f32 = jnp.float32
i32 = jnp.int32


def _np_threefry_uniform(seed, n, lo, hi):
    rot = [np.uint32([13, 15, 26, 6]), np.uint32([17, 29, 16, 24])]

    def rotl(v, r):
        return ((v << np.uint32(r)) | (v >> np.uint32(32 - r))).astype(np.uint32)

    idx = np.arange(n, dtype=np.uint64)
    ks = [np.uint32(0), np.uint32(seed),
          np.uint32(np.uint32(0) ^ np.uint32(seed) ^ np.uint32(0x1BD11BDA))]
    x = [((idx >> np.uint64(32)).astype(np.uint32) + ks[0]).astype(np.uint32),
         ((idx & np.uint64(0xFFFFFFFF)).astype(np.uint32) + ks[1]).astype(np.uint32)]
    for i in range(5):
        for r in rot[i % 2]:
            x[0] = (x[0] + x[1]).astype(np.uint32)
            x[1] = rotl(x[1], r)
            x[1] = (x[1] ^ x[0]).astype(np.uint32)
        x[0] = (x[0] + ks[(i + 1) % 3]).astype(np.uint32)
        x[1] = (x[1] + ks[(i + 2) % 3] + np.uint32(i + 1)).astype(np.uint32)
    bits = x[0] ^ x[1]
    fb = (bits >> np.uint32(9)) | np.uint32(0x3F800000)
    f = fb.view(np.float32) - np.float32(1.0)
    out = f * np.float32(hi - lo) + np.float32(lo)
    return np.maximum(np.float32(lo), out)


def _surf_table():
    u = _np_threefry_uniform(42, NSAMP * 3, -1.0, 1.0).reshape(NSAMP, 3)
    surf = u / np.max(np.abs(u), axis=-1, keepdims=True)
    out = np.zeros((3, NS_PAD), np.float32)
    out[:, :NSAMP] = surf.T
    return out.reshape(-1)


_SURF_T = _surf_table()


def _rsqrt(x):
    i = plsc.bitcast(x, i32)
    y = plsc.bitcast(jnp.int32(0x5F3759DF) - lax.shift_right_logical(i, 1), f32)
    for _ in range(3):
        y = y * (1.5 - 0.5 * x * y * y)
    return y


def _sqrt(x):
    return x * _rsqrt(x)


def _qn_rows(pack_v, qn_v):
    qw = pack_v[pl.ds(OFF_QUAT, L)]
    qx = pack_v[pl.ds(OFF_QUAT + L, L)]
    qy = pack_v[pl.ds(OFF_QUAT + 2 * L, L)]
    qz = pack_v[pl.ds(OFF_QUAT + 3 * L, L)]
    s = qw * qw + qx * qx + qy * qy + qz * qz
    n = s * _rsqrt(s)
    inv = 1.0 / (n + 1e-8)
    qn_v[pl.ds(0, L)] = qw * inv
    qn_v[pl.ds(L, L)] = qx * inv
    qn_v[pl.ds(2 * L, L)] = qy * inv
    qn_v[pl.ds(3 * L, L)] = qz * inv


def _p_consts(pack_v, qn_v, pvec):
    w = plsc.load_gather(qn_v, [pvec])
    x = plsc.load_gather(qn_v, [pvec + L])
    y = plsc.load_gather(qn_v, [pvec + 2 * L])
    z = plsc.load_gather(qn_v, [pvec + 3 * L])
    tx = plsc.load_gather(pack_v, [pvec + OFF_TRANS])
    ty = plsc.load_gather(pack_v, [pvec + (OFF_TRANS + L)])
    tz = plsc.load_gather(pack_v, [pvec + (OFF_TRANS + 2 * L)])
    sx = plsc.load_gather(pack_v, [pvec + OFF_SHAPE])
    sy = plsc.load_gather(pack_v, [pvec + (OFF_SHAPE + L)])
    sz = plsc.load_gather(pack_v, [pvec + (OFF_SHAPE + 2 * L)])
    x2, y2, z2 = x + x, y + y, z + z
    xx, yy, zz = x * x2, y * y2, z * z2
    xy, xz, yz = x * y2, x * z2, y * z2
    wx, wy, wz = w * x2, w * y2, w * z2
    r = ((1.0 - (yy + zz), xy - wz, xz + wy),
         (xy + wz, 1.0 - (xx + zz), yz - wx),
         (xz - wy, yz + wx, 1.0 - (xx + yy)))
    return r, (tx, ty, tz), (sx, sy, sz)


_MESH = plsc.VectorSubcoreMesh(
    core_axis_name="c", subcore_axis_name="s", num_cores=2, num_subcores=16
)

_CPARAMS = pltpu.CompilerParams(needs_layout_passes=False)


@functools.partial(
    pl.kernel,
    out_type=jax.ShapeDtypeStruct((B * L,), f32),
    mesh=_MESH,
    compiler_params=_CPARAMS,
    scratch_types=[
        pltpu.VMEM((ROW,), f32),
        pltpu.VMEM((SG_PAD,), f32),
        pltpu.VMEM((4 * L,), f32),
        pltpu.VMEM((L,), f32),
    ],
)
def _sc_part1(pack_hbm, out_hbm, pack_v, tsdf_v, qn_v, out_v):
    b = lax.axis_index("s") * 2 + lax.axis_index("c")
    iota = jnp.arange(L, dtype=i32)

    pltpu.sync_copy(pack_hbm.at[b], pack_v)
    _qn_rows(pack_v, qn_v)

    big_vec = jnp.full((L,), BIG, f32)

    def init_body(ci, carry):
        for k in range(U1):
            tsdf_v[pl.ds((ci * U1 + k) * L, L)] = big_vec
        return carry

    lax.fori_loop(0, N_CH1 // U1, init_body, 0)

    def p1_body(p, carry):
        pvec = jnp.zeros((L,), i32) + p
        iu = jnp.max(plsc.load_gather(pack_v, [pvec + OFF_IU]))

        @pl.when(iu > 0.0)
        def _():
            r, (tx, ty, tz), (sx, sy, sz) = _p_consts(pack_v, qn_v, pvec)

            def body(ci, c2):
                for k in range(U1):
                    base = (ci * U1 + k) * L
                    vx = pack_v[pl.ds(OFF_PTS + base, L)] - tx
                    vy = pack_v[pl.ds(OFF_PTS + SG + base, L)] - ty
                    vz = pack_v[pl.ds(OFF_PTS + 2 * SG + base, L)] - tz
                    lx = r[0][0] * vx + r[1][0] * vy + r[2][0] * vz
                    ly = r[0][1] * vx + r[1][1] * vy + r[2][1] * vz
                    lz = r[0][2] * vx + r[1][2] * vy + r[2][2] * vz
                    dx = jnp.maximum(jnp.abs(lx) - sx, 0.0)
                    dy = jnp.maximum(jnp.abs(ly) - sy, 0.0)
                    dz = jnp.maximum(jnp.abs(lz) - sz, 0.0)
                    t = dx * dx + dy * dy + dz * dz
                    tsdf_v[pl.ds(base, L)] = jnp.minimum(
                        tsdf_v[pl.ds(base, L)], t)
                return c2

            lax.fori_loop(0, N_CH1 // U1, body, 0)

        return carry

    lax.fori_loop(0, P, p1_body, 0)

    def red_body(ci, acc):
        for k in range(U1):
            base = (ci * U1 + k) * L
            v = tsdf_v[pl.ds(base, L)] + EPS
            sq = _sqrt(v)
            valid = (base + iota) < SG
            acc = acc + jnp.where(valid, sq, 0.0)
        return acc

    acc1 = lax.fori_loop(0, N_CH1 // U1, red_body, jnp.zeros((L,), f32))
    out_v[:] = acc1 * (1.0 / (B * SG))
    pltpu.sync_copy(out_v, out_hbm.at[pl.ds(b * L, L)])


@functools.partial(
    pl.kernel,
    out_type=jax.ShapeDtypeStruct((B * L,), f32),
    mesh=_MESH,
    compiler_params=_CPARAMS,
    scratch_types=[
        pltpu.VMEM((NCELL * 3,), f32),
        pltpu.VMEM((ROW,), f32),
        pltpu.VMEM((3 * NS_PAD,), f32),
        pltpu.VMEM((4 * L,), f32),
        pltpu.VMEM((L,), f32),
        pltpu.VMEM((L,), f32),
        pltpu.SemaphoreType.DMA,
    ],
)
def _sc_part2(pack_hbm, cp_hbm, surf_hbm, out_hbm, cp_v, pack_v, surf_v,
              qn_v, acc2_v, out_v, cp_sem):
    b = lax.axis_index("s") * 2 + lax.axis_index("c")
    iota = jnp.arange(L, dtype=i32)

    cp_copy = pltpu.async_copy(cp_hbm.at[b], cp_v, cp_sem)
    pltpu.sync_copy(pack_hbm.at[b], pack_v)
    pltpu.sync_copy(surf_hbm, surf_v)
    _qn_rows(pack_v, qn_v)

    cp_copy.wait()
    acc2_v[:] = jnp.zeros((L,), f32)
    sqrt_eps = _sqrt(jnp.full((L,), EPS, f32))
    onehot0 = jnp.where(iota == 0, 1.0, 0.0).astype(f32)

    def p2_body(p, carry):
        pvec = jnp.zeros((L,), i32) + p
        iu = jnp.max(plsc.load_gather(pack_v, [pvec + OFF_IU]))

        @pl.when(iu > 0.0)
        def _():
            r, (tx, ty, tz), (sx, sy, sz) = _p_consts(pack_v, qn_v, pvec)

            def body(ci, acc):
                for k in range(2):
                    base = (ci * 2 + k) * L
                    plx = surf_v[pl.ds(base, L)] * sx
                    ply = surf_v[pl.ds(NS_PAD + base, L)] * sy
                    plz = surf_v[pl.ds(2 * NS_PAD + base, L)] * sz
                    px = r[0][0] * plx + r[0][1] * ply + r[0][2] * plz + tx
                    py = r[1][0] * plx + r[1][1] * ply + r[1][2] * plz + ty
                    pz = r[2][0] * plx + r[2][1] * ply + r[2][2] * plz + tz
                    gx = jnp.clip(((px + 0.5) * float(GRID)).astype(i32),
                                  0, GRID - 1)
                    gy = jnp.clip(((py + 0.5) * float(GRID)).astype(i32),
                                  0, GRID - 1)
                    gz = jnp.clip(((pz + 0.5) * float(GRID)).astype(i32),
                                  0, GRID - 1)
                    lin = gx * 3072 + gy * 32 + gz
                    cx = plsc.load_gather(cp_v, [lin])
                    cy = plsc.load_gather(cp_v, [lin + 1024])
                    cz = plsc.load_gather(cp_v, [lin + 2048])
                    ex, ey, ez = px - cx, py - cy, pz - cz
                    d2 = ex * ex + ey * ey + ez * ez + EPS
                    dist = _sqrt(d2)
                    valid = (base + iota) < NSAMP
                    acc = acc + jnp.where(valid, dist, 0.0)
                return acc

            acc_p = lax.fori_loop(0, N_CH2 // 2, body, jnp.zeros((L,), f32))
            acc2_v[:] = acc2_v[:] + acc_p

        @pl.when(iu <= 0.0)
        def _():
            acc2_v[:] = acc2_v[:] + onehot0 * (float(NSAMP) * sqrt_eps)

        return carry

    lax.fori_loop(0, P, p2_body, 0)

    out_v[:] = acc2_v[:] * (1.0 / (B * P * NSAMP))
    pltpu.sync_copy(out_v, out_hbm.at[pl.ds(b * L, L)])


def kernel(shape_rlt, trans_rlt, quat_rlt, CP, batchSamplepoint, inUse):
    pack = jnp.concatenate(
        [
            shape_rlt.transpose(0, 2, 1).reshape(B, 3 * P),
            trans_rlt.transpose(0, 2, 1).reshape(B, 3 * P),
            quat_rlt.transpose(0, 2, 1).reshape(B, 4 * P),
            inUse.astype(f32),
            batchSamplepoint.transpose(0, 2, 1).reshape(B, 3 * SG),
            jnp.zeros((B, ROW - OFF_PTS - 3 * SG), f32),
        ],
        axis=1,
    )
    cp = CP.transpose(0, 1, 4, 2, 3).reshape(B, NCELL * 3)
    out1 = _sc_part1(pack)
    out2 = _sc_part2(pack, cp, jnp.asarray(_SURF_T))
    return jnp.sum(out1) + jnp.sum(out2)


# speedup vs baseline: 1.1080x; 1.1080x over previous
import functools

import numpy as np

import jax
import jax.numpy as jnp
from jax import lax
from jax.experimental import pallas as pl
from jax.experimental.pallas import tpu as pltpu
from jax.experimental.pallas import tpu_sc as plsc

B = 32
P = 16
SG = 1000
NSAMP = 150
GRID = 32
EPS = 1e-12
BIG = 1e4

L = 16
SG_PAD = 1008
NS_PAD = 160
U1 = 7
N_CH1 = SG_PAD // L
N_CH2 = NS_PAD // L
NCELL = GRID * GRID * GRID

OFF_SHAPE = 0
OFF_TRANS = 48
OFF_QUAT = 96
OFF_IU = 160
OFF_PTS = 176
ROW = 3200

f32 = jnp.float32
i32 = jnp.int32


def _np_threefry_uniform(seed, n, lo, hi):
    rot = [np.uint32([13, 15, 26, 6]), np.uint32([17, 29, 16, 24])]

    def rotl(v, r):
        return ((v << np.uint32(r)) | (v >> np.uint32(32 - r))).astype(np.uint32)

    idx = np.arange(n, dtype=np.uint64)
    ks = [np.uint32(0), np.uint32(seed),
          np.uint32(np.uint32(0) ^ np.uint32(seed) ^ np.uint32(0x1BD11BDA))]
    x = [((idx >> np.uint64(32)).astype(np.uint32) + ks[0]).astype(np.uint32),
         ((idx & np.uint64(0xFFFFFFFF)).astype(np.uint32) + ks[1]).astype(np.uint32)]
    for i in range(5):
        for r in rot[i % 2]:
            x[0] = (x[0] + x[1]).astype(np.uint32)
            x[1] = rotl(x[1], r)
            x[1] = (x[1] ^ x[0]).astype(np.uint32)
        x[0] = (x[0] + ks[(i + 1) % 3]).astype(np.uint32)
        x[1] = (x[1] + ks[(i + 2) % 3] + np.uint32(i + 1)).astype(np.uint32)
    bits = x[0] ^ x[1]
    fb = (bits >> np.uint32(9)) | np.uint32(0x3F800000)
    f = fb.view(np.float32) - np.float32(1.0)
    out = f * np.float32(hi - lo) + np.float32(lo)
    return np.maximum(np.float32(lo), out)


def _surf_table():
    u = _np_threefry_uniform(42, NSAMP * 3, -1.0, 1.0).reshape(NSAMP, 3)
    surf = u / np.max(np.abs(u), axis=-1, keepdims=True)
    out = np.zeros((3, NS_PAD), np.float32)
    out[:, :NSAMP] = surf.T
    return out.reshape(-1)


_SURF_T = _surf_table()


def _rsqrt(x):
    i = plsc.bitcast(x, i32)
    y = plsc.bitcast(jnp.int32(0x5F3759DF) - lax.shift_right_logical(i, 1), f32)
    for _ in range(3):
        y = y * (1.5 - 0.5 * x * y * y)
    return y


def _sqrt(x):
    return x * _rsqrt(x)


def _qn_rows(pack_v, qn_v):
    qw = pack_v[pl.ds(OFF_QUAT, L)]
    qx = pack_v[pl.ds(OFF_QUAT + L, L)]
    qy = pack_v[pl.ds(OFF_QUAT + 2 * L, L)]
    qz = pack_v[pl.ds(OFF_QUAT + 3 * L, L)]
    s = qw * qw + qx * qx + qy * qy + qz * qz
    n = s * _rsqrt(s)
    inv = 1.0 / (n + 1e-8)
    qn_v[pl.ds(0, L)] = qw * inv
    qn_v[pl.ds(L, L)] = qx * inv
    qn_v[pl.ds(2 * L, L)] = qy * inv
    qn_v[pl.ds(3 * L, L)] = qz * inv


def _p_consts(pack_v, qn_v, pvec):
    w = plsc.load_gather(qn_v, [pvec])
    x = plsc.load_gather(qn_v, [pvec + L])
    y = plsc.load_gather(qn_v, [pvec + 2 * L])
    z = plsc.load_gather(qn_v, [pvec + 3 * L])
    tx = plsc.load_gather(pack_v, [pvec + OFF_TRANS])
    ty = plsc.load_gather(pack_v, [pvec + (OFF_TRANS + L)])
    tz = plsc.load_gather(pack_v, [pvec + (OFF_TRANS + 2 * L)])
    sx = plsc.load_gather(pack_v, [pvec + OFF_SHAPE])
    sy = plsc.load_gather(pack_v, [pvec + (OFF_SHAPE + L)])
    sz = plsc.load_gather(pack_v, [pvec + (OFF_SHAPE + 2 * L)])
    x2, y2, z2 = x + x, y + y, z + z
    xx, yy, zz = x * x2, y * y2, z * z2
    xy, xz, yz = x * y2, x * z2, y * z2
    wx, wy, wz = w * x2, w * y2, w * z2
    r = ((1.0 - (yy + zz), xy - wz, xz + wy),
         (xy + wz, 1.0 - (xx + zz), yz - wx),
         (xz - wy, yz + wx, 1.0 - (xx + yy)))
    return r, (tx, ty, tz), (sx, sy, sz)


_MESH = plsc.VectorSubcoreMesh(
    core_axis_name="c", subcore_axis_name="s", num_cores=2, num_subcores=16
)

_CPARAMS = pltpu.CompilerParams(needs_layout_passes=False)


@functools.partial(
    pl.kernel,
    out_type=jax.ShapeDtypeStruct((B * 2 * L,), f32),
    mesh=_MESH,
    compiler_params=_CPARAMS,
    scratch_types=[
        pltpu.VMEM((NCELL * 3,), f32),
        pltpu.VMEM((ROW,), f32),
        pltpu.VMEM((SG_PAD,), f32),
        pltpu.VMEM((3 * NS_PAD,), f32),
        pltpu.VMEM((4 * L,), f32),
        pltpu.VMEM((L,), f32),
        pltpu.VMEM((2 * L,), f32),
        pltpu.SemaphoreType.DMA,
    ],
)
def _sc_kernel(pack_hbm, cp_hbm, surf_hbm, out_hbm, cp_v, pack_v, tsdf_v,
               surf_v, qn_v, acc2_v, out_v, cp_sem):
    b = lax.axis_index("s") * 2 + lax.axis_index("c")
    iota = jnp.arange(L, dtype=i32)

    cp_copy = pltpu.async_copy(cp_hbm.at[b], cp_v, cp_sem)
    pltpu.sync_copy(pack_hbm.at[b], pack_v)
    pltpu.sync_copy(surf_hbm, surf_v)
    _qn_rows(pack_v, qn_v)

    big_vec = jnp.full((L,), BIG, f32)

    def init_body(ci, carry):
        for k in range(U1):
            tsdf_v[pl.ds((ci * U1 + k) * L, L)] = big_vec
        return carry

    lax.fori_loop(0, N_CH1 // U1, init_body, 0)

    def p1_body(p, carry):
        pvec = jnp.zeros((L,), i32) + p
        iu = jnp.max(plsc.load_gather(pack_v, [pvec + OFF_IU]))

        @pl.when(iu > 0.0)
        def _():
            r, (tx, ty, tz), (sx, sy, sz) = _p_consts(pack_v, qn_v, pvec)

            def body(ci, c2):
                for k in range(U1):
                    base = (ci * U1 + k) * L
                    vx = pack_v[pl.ds(OFF_PTS + base, L)] - tx
                    vy = pack_v[pl.ds(OFF_PTS + SG + base, L)] - ty
                    vz = pack_v[pl.ds(OFF_PTS + 2 * SG + base, L)] - tz
                    lx = r[0][0] * vx + r[1][0] * vy + r[2][0] * vz
                    ly = r[0][1] * vx + r[1][1] * vy + r[2][1] * vz
                    lz = r[0][2] * vx + r[1][2] * vy + r[2][2] * vz
                    dx = jnp.maximum(jnp.abs(lx) - sx, 0.0)
                    dy = jnp.maximum(jnp.abs(ly) - sy, 0.0)
                    dz = jnp.maximum(jnp.abs(lz) - sz, 0.0)
                    t = dx * dx + dy * dy + dz * dz
                    tsdf_v[pl.ds(base, L)] = jnp.minimum(
                        tsdf_v[pl.ds(base, L)], t)
                return c2

            lax.fori_loop(0, N_CH1 // U1, body, 0)

        return carry

    lax.fori_loop(0, P, p1_body, 0)

    def red_body(ci, acc):
        for k in range(U1):
            base = (ci * U1 + k) * L
            v = tsdf_v[pl.ds(base, L)] + EPS
            sq = _sqrt(v)
            valid = (base + iota) < SG
            acc = acc + jnp.where(valid, sq, 0.0)
        return acc

    acc1 = lax.fori_loop(0, N_CH1 // U1, red_body, jnp.zeros((L,), f32))

    cp_copy.wait()
    acc2_v[:] = jnp.zeros((L,), f32)
    sqrt_eps = _sqrt(jnp.full((L,), EPS, f32))
    onehot0 = jnp.where(iota == 0, 1.0, 0.0).astype(f32)

    def p2_body(p, carry):
        pvec = jnp.zeros((L,), i32) + p
        iu = jnp.max(plsc.load_gather(pack_v, [pvec + OFF_IU]))

        @pl.when(iu > 0.0)
        def _():
            r, (tx, ty, tz), (sx, sy, sz) = _p_consts(pack_v, qn_v, pvec)

            def body(ci, acc):
                for k in range(2):
                    base = (ci * 2 + k) * L
                    plx = surf_v[pl.ds(base, L)] * sx
                    ply = surf_v[pl.ds(NS_PAD + base, L)] * sy
                    plz = surf_v[pl.ds(2 * NS_PAD + base, L)] * sz
                    px = r[0][0] * plx + r[0][1] * ply + r[0][2] * plz + tx
                    py = r[1][0] * plx + r[1][1] * ply + r[1][2] * plz + ty
                    pz = r[2][0] * plx + r[2][1] * ply + r[2][2] * plz + tz
                    gx = jnp.clip(((px + 0.5) * float(GRID)).astype(i32),
                                  0, GRID - 1)
                    gy = jnp.clip(((py + 0.5) * float(GRID)).astype(i32),
                                  0, GRID - 1)
                    gz = jnp.clip(((pz + 0.5) * float(GRID)).astype(i32),
                                  0, GRID - 1)
                    lin = gx * 3072 + gy * 32 + gz
                    cx = plsc.load_gather(cp_v, [lin])
                    cy = plsc.load_gather(cp_v, [lin + 1024])
                    cz = plsc.load_gather(cp_v, [lin + 2048])
                    ex, ey, ez = px - cx, py - cy, pz - cz
                    d2 = ex * ex + ey * ey + ez * ez + EPS
                    dist = _sqrt(d2)
                    valid = (base + iota) < NSAMP
                    acc = acc + jnp.where(valid, dist, 0.0)
                return acc

            acc_p = lax.fori_loop(0, N_CH2 // 2, body, jnp.zeros((L,), f32))
            acc2_v[:] = acc2_v[:] + acc_p

        @pl.when(iu <= 0.0)
        def _():
            acc2_v[:] = acc2_v[:] + onehot0 * (float(NSAMP) * sqrt_eps)

        return carry

    lax.fori_loop(0, P, p2_body, 0)

    out_v[pl.ds(0, L)] = acc1 * (1.0 / (B * SG))
    out_v[pl.ds(L, L)] = acc2_v[:] * (1.0 / (B * P * NSAMP))
    pltpu.sync_copy(out_v, out_hbm.at[pl.ds(b * 2 * L, 2 * L)])


def kernel(shape_rlt, trans_rlt, quat_rlt, CP, batchSamplepoint, inUse):
    pack = jnp.concatenate(
        [
            shape_rlt.transpose(0, 2, 1).reshape(B, 3 * P),
            trans_rlt.transpose(0, 2, 1).reshape(B, 3 * P),
            quat_rlt.transpose(0, 2, 1).reshape(B, 4 * P),
            inUse.astype(f32),
            batchSamplepoint.transpose(0, 2, 1).reshape(B, 3 * SG),
            jnp.zeros((B, ROW - OFF_PTS - 3 * SG), f32),
        ],
        axis=1,
    )
    cp = CP.transpose(0, 1, 4, 2, 3).reshape(B, NCELL * 3)
    out = _sc_kernel(pack, cp, jnp.asarray(_SURF_T))
    return jnp.sum(out)
